# baseline (speedup 1.0000x reference)
import functools

import jax
import jax.numpy as jnp
from jax import lax
from jax.experimental import pallas as pl
from jax.experimental.pallas import tpu as pltpu
from jax.experimental.pallas import tpu_sc as plsc

N = 4096
NB = 8192
E = 32
H = 256
DEPTH = 10
BATCH = 64
NUM_ATOM_TYPES = 200
NUM_BOND_TYPES = 10

_ROWS = 512
_NBLK = N // _ROWS


def _store_h2(h2_ref, xT):
    f32 = jnp.float32
    f8 = jnp.float8_e4m3fn
    hN = jnp.transpose(xT)
    t0 = hN.astype(f8)
    r0 = hN - t0.astype(f32)
    t1 = (r0 * 256.0).astype(f8)
    r1 = r0 - t1.astype(f32) * (1.0 / 256.0)
    t2 = (r1 * 65536.0).astype(f8)
    h2_ref[...] = jnp.concatenate([t0, t1, t2], axis=1)


def _mpn_body(adj_ref, af_ref, bi_ref, atom_embT_ref,
              msgW_ref, mbc_ref, Wih_ref, Whh_ref, bihc_ref, bhhc_ref,
              poolW_ref, pbc_ref,
              hT_out_ref, graphT_ref,
              mask_ref, scale_ref, flag_ref, bufT_ref, h2_ref, agg_ref):
    l = pl.program_id(0)
    b = pl.program_id(1)
    f32 = jnp.float32
    bf16 = jnp.bfloat16
    f8 = jnp.float8_e4m3fn

    @pl.when(jnp.logical_and(l == 0, b == 0))
    def _init():
        af = af_ref[...]
        iota = jax.lax.broadcasted_iota(jnp.int32, (NUM_ATOM_TYPES, N), 0)
        ohT = (iota == af).astype(f32)
        h0T = jnp.dot(atom_embT_ref[...], ohT, preferred_element_type=f32)
        bufT_ref[0] = h0T
        _store_h2(h2_ref, h0T)

    @pl.when(l == 1)
    def _convert():
        a = adj_ref[...]
        m = a != 0
        mask_ref[pl.ds(b * _ROWS, _ROWS), :] = m.astype(f8)
        deg = jnp.sum(m.astype(f32), axis=1, keepdims=True)
        has = deg > 0.0
        safe = jnp.where(has, deg, 1.0)
        sc = jnp.where(has, 1.0 / safe, 0.0)
        scale_ref[:, pl.ds(b * _ROWS, _ROWS)] = jnp.transpose(sc)
        flag_ref[:, pl.ds(b * _ROWS, _ROWS)] = jnp.transpose(
            has.astype(f32))

    @pl.when(l > 0)
    def _layer():
        mblk = mask_ref[pl.ds(b * _ROWS, _ROWS), :]
        acc = jnp.dot(mblk, h2_ref[...], preferred_element_type=f32)
        agg_ref[pl.ds(b * _ROWS, _ROWS), :] = acc

        @pl.when(b == _NBLK - 1)
        def _gates():
            Wih = Wih_ref[0]
            W = msgW_ref[0]
            A_T = jnp.dot(Wih, W[:, :E], preferred_element_type=f32)
            B_T = jnp.dot(Wih, W[:, E:], preferred_element_type=f32)
            top = jnp.concatenate([A_T, B_T], axis=1)
            bot = jnp.concatenate([Whh_ref[0], jnp.zeros((3 * E, E), f32)],
                                  axis=1)
            CT = jnp.concatenate([top, bot], axis=0)
            CThi = CT.astype(bf16)
            CTlo = (CT - CThi.astype(f32)).astype(bf16)
            CT2 = jnp.concatenate([CThi, CTlo], axis=0)
            rowT = jnp.dot(Wih, mbc_ref[0], preferred_element_type=f32)

            aggT = jnp.transpose(agg_ref[...])
            agg1 = (aggT[:E] + aggT[E:2 * E] * (1.0 / 256.0)
                    + aggT[2 * E:] * (1.0 / 65536.0))
            aggsT = agg1 * scale_ref[...]
            hT = bufT_ref[(l - 1) % 2]
            XT = jnp.concatenate([hT, aggsT], axis=0)
            XThi = XT.astype(bf16)
            XTlo = (XT - XThi.astype(f32)).astype(bf16)
            G1 = jnp.dot(CT2, XThi, preferred_element_type=f32)
            G2 = jnp.dot(CThi, XTlo, preferred_element_type=f32)
            GT = G1[:6 * E] + G1[6 * E:] + G2
            giT = flag_ref[...] * (GT[:3 * E] + rowT) + bihc_ref[0]
            ghT = GT[3 * E:] + bhhc_ref[0]
            rT = jax.nn.sigmoid(giT[:E] + ghT[:E])
            zT = jax.nn.sigmoid(giT[E:2 * E] + ghT[E:2 * E])
            nT = jnp.tanh(giT[2 * E:] + rT * ghT[2 * E:])
            hnT = (1.0 - zT) * nT + zT * hT
            bufT_ref[l % 2] = hnT
            _store_h2(h2_ref, hnT)

            @pl.when(l == DEPTH)
            def _emit():
                hT_out_ref[...] = hnT
                bi = bi_ref[...]
                iota = jax.lax.broadcasted_iota(jnp.int32, (N, BATCH), 1)
                PT = (iota == bi).astype(f32)
                countsT = jnp.sum(PT, axis=0, keepdims=True)
                sumsT = jnp.dot(hnT, PT, preferred_element_type=f32)
                inv = jnp.where(countsT > 0.0,
                                1.0 / jnp.where(countsT > 0.0, countsT, 1.0),
                                0.0)
                meansT = sumsT * inv
                graphT_ref[...] = (jnp.dot(poolW_ref[...], meansT,
                                           preferred_element_type=f32)
                                   + pbc_ref[...])


_SC_NC = 2
_SC_NS = 16
_B_PER_W = NB // (_SC_NC * _SC_NS)
_IDX_CHUNK = 128
_DPAD = 128


def _bond_gather_sc(table_hbm, idx_hbm, out_hbm, idx_v, rows_v, sem):
    wid = lax.axis_index("s") * _SC_NC + lax.axis_index("c")
    base = wid * _B_PER_W
    for j in range(_B_PER_W // _IDX_CHUNK):
        off = base + j * _IDX_CHUNK
        pltpu.sync_copy(idx_hbm.at[pl.ds(off, _IDX_CHUNK)], idx_v)
        pltpu.async_copy(table_hbm.at[idx_v], rows_v, sem).wait()
        pltpu.sync_copy(rows_v, out_hbm.at[pl.ds(off, _IDX_CHUNK)])


def kernel(atom_features, bond_features, adjacency_matrix, batch_indices,
           atom_emb, bond_emb, msg_W, msg_b,
           gru_Wih, gru_Whh, gru_bih, gru_bhh, pool_W, pool_b):
    af = atom_features.reshape(1, N).astype(jnp.int32)
    bf1 = bond_features.reshape(NB).astype(jnp.int32)
    bi = batch_indices.reshape(N, 1).astype(jnp.int32)
    atom_embT = atom_emb.T
    mbc = msg_b.reshape(DEPTH, H, 1)
    bihc = gru_bih.reshape(DEPTH, 3 * E, 1)
    bhhc = gru_bhh.reshape(DEPTH, 3 * E, 1)
    pbc = pool_b.reshape(H, 1)

    def _wmap(nd):
        def im(l, b):
            lw = jnp.maximum(l - 1, 0)
            return (lw,) + (0,) * (nd - 1)
        return im

    def _adj_map(l, b):
        return (jnp.where(l >= 2, _NBLK - 1, jnp.where(l == 1, b, 0)), 0)

    hT_out, graphT = pl.pallas_call(
        _mpn_body,
        grid=(DEPTH + 1, _NBLK),
        in_specs=[
            pl.BlockSpec((_ROWS, N), _adj_map),
            pl.BlockSpec((1, N), lambda l, b: (0, 0)),
            pl.BlockSpec((N, 1), lambda l, b: (0, 0)),
            pl.BlockSpec((E, NUM_ATOM_TYPES), lambda l, b: (0, 0)),
            pl.BlockSpec((1, H, 2 * E), _wmap(3)),
            pl.BlockSpec((1, H, 1), _wmap(3)),
            pl.BlockSpec((1, 3 * E, H), _wmap(3)),
            pl.BlockSpec((1, 3 * E, E), _wmap(3)),
            pl.BlockSpec((1, 3 * E, 1), _wmap(3)),
            pl.BlockSpec((1, 3 * E, 1), _wmap(3)),
            pl.BlockSpec((H, E), lambda l, b: (0, 0)),
            pl.BlockSpec((H, 1), lambda l, b: (0, 0)),
        ],
        out_specs=[
            pl.BlockSpec((E, N), lambda l, b: (0, 0)),
            pl.BlockSpec((H, BATCH), lambda l, b: (0, 0)),
        ],
        out_shape=[
            jax.ShapeDtypeStruct((E, N), jnp.float32),
            jax.ShapeDtypeStruct((H, BATCH), jnp.float32),
        ],
        scratch_shapes=[
            pltpu.VMEM((N, N), jnp.float8_e4m3fn),
            pltpu.VMEM((1, N), jnp.float32),
            pltpu.VMEM((1, N), jnp.float32),
            pltpu.VMEM((2, E, N), jnp.float32),
            pltpu.VMEM((N, 3 * E), jnp.float8_e4m3fn),
            pltpu.VMEM((N, 3 * E), jnp.float32),
        ],
    )(adjacency_matrix, af, bi, atom_embT, msg_W, mbc, gru_Wih, gru_Whh,
      bihc, bhhc, pool_W, pbc)

    bond_kernel = functools.partial(
        pl.kernel,
        mesh=plsc.VectorSubcoreMesh(core_axis_name="c", subcore_axis_name="s"),
        out_type=jax.ShapeDtypeStruct((NB, _DPAD), jnp.float32),
        scratch_types=[
            pltpu.VMEM((_IDX_CHUNK,), jnp.int32),
            pltpu.VMEM((_IDX_CHUNK, _DPAD), jnp.float32),
            pltpu.SemaphoreType.DMA,
        ],
    )(_bond_gather_sc)
    bond_emb_pad = jnp.pad(bond_emb, ((0, 0), (0, _DPAD - E)))
    bond_out = bond_kernel(bond_emb_pad, bf1)[:, :E]

    return (hT_out.T, bond_out, graphT.T)


# speedup vs baseline: 1.2379x; 1.2379x over previous
import jax
import jax.numpy as jnp
from jax.experimental import pallas as pl
from jax.experimental.pallas import tpu as pltpu

N = 4096
NB = 8192
E = 32
H = 256
DEPTH = 10
BATCH = 64
NUM_ATOM_TYPES = 200
NUM_BOND_TYPES = 10

_ROWS = 512
_NBLK = N // _ROWS


def _store_h2(h2_ref, xT):
    f32 = jnp.float32
    f8 = jnp.float8_e4m3fn
    hN = jnp.transpose(xT)
    t0 = hN.astype(f8)
    r0 = hN - t0.astype(f32)
    t1 = (r0 * 256.0).astype(f8)
    r1 = r0 - t1.astype(f32) * (1.0 / 256.0)
    t2 = (r1 * 65536.0).astype(f8)
    h2_ref[...] = jnp.concatenate([t0, t1, t2], axis=1)


def _mpn_body(adj_ref, af_ref, bi_ref, atom_embT_ref,
              msgW_ref, mbc_ref, Wih_ref, Whh_ref, bihc_ref, bhhc_ref,
              poolW_ref, pbc_ref,
              hT_out_ref, graphT_ref,
              mask_ref, scale_ref, flag_ref, bufT_ref, h2_ref, agg_ref):
    l = pl.program_id(0)
    b = pl.program_id(1)
    f32 = jnp.float32
    bf16 = jnp.bfloat16
    f8 = jnp.float8_e4m3fn

    @pl.when(jnp.logical_and(l == 0, b == 0))
    def _init():
        af = af_ref[...]
        iota = jax.lax.broadcasted_iota(jnp.int32, (NUM_ATOM_TYPES, N), 0)
        ohT = (iota == af).astype(f32)
        h0T = jnp.dot(atom_embT_ref[...], ohT, preferred_element_type=f32)
        bufT_ref[0] = h0T
        _store_h2(h2_ref, h0T)

    @pl.when(l == 1)
    def _convert():
        a = adj_ref[...]
        m = a != 0
        mask_ref[pl.ds(b * _ROWS, _ROWS), :] = m.astype(f8)
        deg = jnp.sum(m.astype(f32), axis=1, keepdims=True)
        has = deg > 0.0
        safe = jnp.where(has, deg, 1.0)
        sc = jnp.where(has, 1.0 / safe, 0.0)
        scale_ref[:, pl.ds(b * _ROWS, _ROWS)] = jnp.transpose(sc)
        flag_ref[:, pl.ds(b * _ROWS, _ROWS)] = jnp.transpose(
            has.astype(f32))

    @pl.when(l > 0)
    def _layer():
        mblk = mask_ref[pl.ds(b * _ROWS, _ROWS), :]
        acc = jnp.dot(mblk, h2_ref[...], preferred_element_type=f32)
        agg_ref[pl.ds(b * _ROWS, _ROWS), :] = acc

        @pl.when(b == _NBLK - 1)
        def _gates():
            Wih = Wih_ref[0]
            W = msgW_ref[0]
            A_T = jnp.dot(Wih, W[:, :E], preferred_element_type=f32)
            B_T = jnp.dot(Wih, W[:, E:], preferred_element_type=f32)
            top = jnp.concatenate([A_T, B_T], axis=1)
            bot = jnp.concatenate([Whh_ref[0], jnp.zeros((3 * E, E), f32)],
                                  axis=1)
            CT = jnp.concatenate([top, bot], axis=0)
            CThi = CT.astype(bf16)
            CTlo = (CT - CThi.astype(f32)).astype(bf16)
            CT2 = jnp.concatenate([CThi, CTlo], axis=0)
            rowT = jnp.dot(Wih, mbc_ref[0], preferred_element_type=f32)

            aggT = jnp.transpose(agg_ref[...])
            agg1 = (aggT[:E] + aggT[E:2 * E] * (1.0 / 256.0)
                    + aggT[2 * E:] * (1.0 / 65536.0))
            aggsT = agg1 * scale_ref[...]
            hT = bufT_ref[(l - 1) % 2]
            XT = jnp.concatenate([hT, aggsT], axis=0)
            XThi = XT.astype(bf16)
            XTlo = (XT - XThi.astype(f32)).astype(bf16)
            G1 = jnp.dot(CT2, XThi, preferred_element_type=f32)
            G2 = jnp.dot(CThi, XTlo, preferred_element_type=f32)
            GT = G1[:6 * E] + G1[6 * E:] + G2
            giT = flag_ref[...] * (GT[:3 * E] + rowT) + bihc_ref[0]
            ghT = GT[3 * E:] + bhhc_ref[0]
            rT = jax.nn.sigmoid(giT[:E] + ghT[:E])
            zT = jax.nn.sigmoid(giT[E:2 * E] + ghT[E:2 * E])
            nT = jnp.tanh(giT[2 * E:] + rT * ghT[2 * E:])
            hnT = (1.0 - zT) * nT + zT * hT
            bufT_ref[l % 2] = hnT
            _store_h2(h2_ref, hnT)

            @pl.when(l == DEPTH)
            def _emit():
                hT_out_ref[...] = hnT
                bi = bi_ref[...]
                iota = jax.lax.broadcasted_iota(jnp.int32, (N, BATCH), 1)
                PT = (iota == bi).astype(f32)
                countsT = jnp.sum(PT, axis=0, keepdims=True)
                sumsT = jnp.dot(hnT, PT, preferred_element_type=f32)
                inv = jnp.where(countsT > 0.0,
                                1.0 / jnp.where(countsT > 0.0, countsT, 1.0),
                                0.0)
                meansT = sumsT * inv
                graphT_ref[...] = (jnp.dot(poolW_ref[...], meansT,
                                           preferred_element_type=f32)
                                   + pbc_ref[...])


def _bond_body(bf_ref, bond_emb_ref, out_ref):
    bfi = bf_ref[...]
    iota = jax.lax.broadcasted_iota(jnp.int32, (NB, NUM_BOND_TYPES), 1)
    oh = (bfi == iota).astype(jnp.float32)
    out_ref[...] = jnp.dot(oh, bond_emb_ref[...],
                           preferred_element_type=jnp.float32)


def kernel(atom_features, bond_features, adjacency_matrix, batch_indices,
           atom_emb, bond_emb, msg_W, msg_b,
           gru_Wih, gru_Whh, gru_bih, gru_bhh, pool_W, pool_b):
    af = atom_features.reshape(1, N).astype(jnp.int32)
    bf = bond_features.reshape(NB, 1).astype(jnp.int32)
    bi = batch_indices.reshape(N, 1).astype(jnp.int32)
    atom_embT = atom_emb.T
    mbc = msg_b.reshape(DEPTH, H, 1)
    bihc = gru_bih.reshape(DEPTH, 3 * E, 1)
    bhhc = gru_bhh.reshape(DEPTH, 3 * E, 1)
    pbc = pool_b.reshape(H, 1)

    def _wmap(nd):
        def im(l, b):
            lw = jnp.maximum(l - 1, 0)
            return (lw,) + (0,) * (nd - 1)
        return im

    def _adj_map(l, b):
        return (jnp.where(l >= 2, _NBLK - 1, jnp.where(l == 1, b, 0)), 0)

    hT_out, graphT = pl.pallas_call(
        _mpn_body,
        grid=(DEPTH + 1, _NBLK),
        in_specs=[
            pl.BlockSpec((_ROWS, N), _adj_map),
            pl.BlockSpec((1, N), lambda l, b: (0, 0)),
            pl.BlockSpec((N, 1), lambda l, b: (0, 0)),
            pl.BlockSpec((E, NUM_ATOM_TYPES), lambda l, b: (0, 0)),
            pl.BlockSpec((1, H, 2 * E), _wmap(3)),
            pl.BlockSpec((1, H, 1), _wmap(3)),
            pl.BlockSpec((1, 3 * E, H), _wmap(3)),
            pl.BlockSpec((1, 3 * E, E), _wmap(3)),
            pl.BlockSpec((1, 3 * E, 1), _wmap(3)),
            pl.BlockSpec((1, 3 * E, 1), _wmap(3)),
            pl.BlockSpec((H, E), lambda l, b: (0, 0)),
            pl.BlockSpec((H, 1), lambda l, b: (0, 0)),
        ],
        out_specs=[
            pl.BlockSpec((E, N), lambda l, b: (0, 0)),
            pl.BlockSpec((H, BATCH), lambda l, b: (0, 0)),
        ],
        out_shape=[
            jax.ShapeDtypeStruct((E, N), jnp.float32),
            jax.ShapeDtypeStruct((H, BATCH), jnp.float32),
        ],
        scratch_shapes=[
            pltpu.VMEM((N, N), jnp.float8_e4m3fn),
            pltpu.VMEM((1, N), jnp.float32),
            pltpu.VMEM((1, N), jnp.float32),
            pltpu.VMEM((2, E, N), jnp.float32),
            pltpu.VMEM((N, 3 * E), jnp.float8_e4m3fn),
            pltpu.VMEM((N, 3 * E), jnp.float32),
        ],
    )(adjacency_matrix, af, bi, atom_embT, msg_W, mbc, gru_Wih, gru_Whh,
      bihc, bhhc, pool_W, pbc)

    bond_out = pl.pallas_call(
        _bond_body,
        out_shape=jax.ShapeDtypeStruct((NB, E), jnp.float32),
    )(bf, bond_emb)

    return (hT_out.T, bond_out, graphT.T)
